# Initial kernel scaffold; baseline (speedup 1.0000x reference)
import jax
import jax.numpy as jnp
from jax.experimental import pallas as pl


def kernel(x_user, x_item, u2i_src, u2i_dst, i2u_src, i2u_dst, n_user, n_item, W_nbr_u2i, W_self_u2i, W_nbr_i2u, W_self_i2u):
    raise NotImplementedError("write your pallas kernel here")


# speedup vs baseline: 3.0173x; 3.0173x over previous
import functools

import jax
import jax.numpy as jnp
from jax import lax
from jax.experimental import pallas as pl
from jax.experimental.pallas import tpu as pltpu
from jax.experimental.pallas import tpu_sc as plsc

NUM_CORES = 2
NUM_SUBCORES = 16
BLK = 128
CHUNK = 8
HALF = 128
PAD_ROWS = 8

_MESH = plsc.VectorSubcoreMesh(
    core_axis_name="c", subcore_axis_name="s",
    num_cores=NUM_CORES, num_subcores=NUM_SUBCORES)


def _stripes(n_dst):
    stripe = ((n_dst + NUM_SUBCORES - 1) // NUM_SUBCORES + 7) // 8 * 8
    return stripe, n_dst - (NUM_SUBCORES - 1) * stripe


def _sc_gather_scatter(x0, x1, esrc2d, edst2d, n_dst):
    n_blocks = esrc2d.shape[0]
    blocks_per_sub = n_blocks // NUM_SUBCORES
    n_chunks = blocks_per_sub // CHUNK
    n_acc = n_dst + PAD_ROWS
    stripe, last_stripe = _stripes(n_dst)

    zf = jnp.zeros((n_dst, HALF), jnp.float32)

    @functools.partial(
        pl.kernel,
        out_type=(
            jax.ShapeDtypeStruct((n_dst, HALF), jnp.float32),
            jax.ShapeDtypeStruct((n_dst, HALF), jnp.float32),
        ),
        mesh=_MESH,
        scratch_types=[
            pltpu.VMEM((CHUNK, BLK), jnp.int32),
            pltpu.VMEM((CHUNK, BLK), jnp.int32),
            pltpu.VMEM((BLK, HALF), jnp.float32),
            pltpu.VMEM((BLK, HALF), jnp.float32),
            pltpu.VMEM_SHARED((n_acc, HALF), jnp.float32),
            pltpu.SemaphoreType.DMA,
            pltpu.SemaphoreType.DMA,
        ],
    )
    def k(x0_hbm, x1_hbm, es_hbm, ed_hbm, zf_hbm,
          agg0_hbm, agg1_hbm,
          idx_s, idx_d, rows0, rows1, acc, sem0, sem1):
        c = lax.axis_index("c")
        s = lax.axis_index("s")

        def on_stripe(fn):
            @pl.when(s < NUM_SUBCORES - 1)
            def _():
                fn(s * stripe, stripe)

            @pl.when(s == NUM_SUBCORES - 1)
            def _():
                fn((NUM_SUBCORES - 1) * stripe, last_stripe)

        blk0 = s * blocks_per_sub
        on_stripe(lambda r0, nr: pltpu.sync_copy(
            zf_hbm.at[pl.ds(r0, nr)], acc.at[pl.ds(r0, nr)]))
        plsc.subcore_barrier()

        bufs = ((rows0, sem0), (rows1, sem1))

        def main_loop(x_hbm):
            @pl.loop(0, n_chunks)
            def _(m):
                pltpu.sync_copy(es_hbm.at[pl.ds(blk0 + m * CHUNK, CHUNK)],
                                idx_s)
                pltpu.sync_copy(ed_hbm.at[pl.ds(blk0 + m * CHUNK, CHUNK)],
                                idx_d)
                pltpu.async_copy(x_hbm.at[idx_s.at[0]], rows0, sem0)
                for j in range(CHUNK):
                    if j + 1 < CHUNK:
                        nrows, nsem = bufs[(j + 1) % 2]
                        pltpu.async_copy(x_hbm.at[idx_s.at[j + 1]],
                                         nrows, nsem)
                    rows, sem = bufs[j % 2]
                    pltpu.make_async_copy(x_hbm.at[idx_s.at[j]],
                                          rows, sem).wait()
                    pltpu.sync_copy(rows, acc.at[idx_d.at[j]], add=True)

        @pl.when(c == 0)
        def _():
            main_loop(x0_hbm)

        @pl.when(c == 1)
        def _():
            main_loop(x1_hbm)

        plsc.subcore_barrier()

        @pl.when(c == 0)
        def _():
            on_stripe(lambda r0, nr: pltpu.sync_copy(
                acc.at[pl.ds(r0, nr)], agg0_hbm.at[pl.ds(r0, nr)]))

        @pl.when(c == 1)
        def _():
            on_stripe(lambda r0, nr: pltpu.sync_copy(
                acc.at[pl.ds(r0, nr)], agg1_hbm.at[pl.ds(r0, nr)]))

    return k(x0, x1, esrc2d, edst2d, zf)


def _sc_degree(edst2d, n_dst):
    n_blocks = edst2d.shape[0]
    blocks_per_w = n_blocks // (NUM_CORES * NUM_SUBCORES)
    n_chunks = blocks_per_w // CHUNK
    n_acc = n_dst + PAD_ROWS
    stripe, last_stripe = _stripes(n_dst)

    zd = jnp.zeros((n_dst, HALF), jnp.float32)
    ones = jnp.ones((BLK, HALF), jnp.float32)

    @functools.partial(
        pl.kernel,
        out_type=(
            jax.ShapeDtypeStruct((n_dst, HALF), jnp.float32),
            jax.ShapeDtypeStruct((n_dst, HALF), jnp.float32),
        ),
        mesh=_MESH,
        scratch_types=[
            pltpu.VMEM((CHUNK, BLK), jnp.int32),
            pltpu.VMEM((BLK, HALF), jnp.float32),
            pltpu.VMEM_SHARED((n_acc, HALF), jnp.float32),
        ],
    )
    def k(ed_hbm, zd_hbm, ones_hbm, dega_hbm, degb_hbm,
          idx_d, ones_v, dacc):
        c = lax.axis_index("c")
        s = lax.axis_index("s")

        def on_stripe(fn):
            @pl.when(s < NUM_SUBCORES - 1)
            def _():
                fn(s * stripe, stripe)

            @pl.when(s == NUM_SUBCORES - 1)
            def _():
                fn((NUM_SUBCORES - 1) * stripe, last_stripe)

        blk0 = (c * NUM_SUBCORES + s) * blocks_per_w
        on_stripe(lambda r0, nr: pltpu.sync_copy(
            zd_hbm.at[pl.ds(r0, nr)], dacc.at[pl.ds(r0, nr)]))
        pltpu.sync_copy(ones_hbm, ones_v)
        plsc.subcore_barrier()

        @pl.loop(0, n_chunks)
        def _(m):
            pltpu.sync_copy(ed_hbm.at[pl.ds(blk0 + m * CHUNK, CHUNK)], idx_d)
            for j in range(CHUNK):
                pltpu.sync_copy(ones_v, dacc.at[idx_d.at[j]], add=True)

        plsc.subcore_barrier()

        @pl.when(c == 0)
        def _():
            on_stripe(lambda r0, nr: pltpu.sync_copy(
                dacc.at[pl.ds(r0, nr)], dega_hbm.at[pl.ds(r0, nr)]))

        @pl.when(c == 1)
        def _():
            on_stripe(lambda r0, nr: pltpu.sync_copy(
                dacc.at[pl.ds(r0, nr)], degb_hbm.at[pl.ds(r0, nr)]))

    return k(edst2d, zd, ones)


def _tc_mean_matmul(agg0, agg1, dega, degb, x_dst, w_nbr, w_self):
    n, d = x_dst.shape
    blk = 1000
    wn0 = w_nbr[:HALF]
    wn1 = w_nbr[HALF:]

    def body(a0_ref, a1_ref, da_ref, db_ref, x_ref,
             wn0_ref, wn1_ref, ws_ref, o_ref):
        deg = da_ref[:, 0:1] + db_ref[:, 0:1]
        r = 1.0 / jnp.maximum(deg, 1.0)
        o_ref[...] = (
            jnp.dot(a0_ref[...] * r, wn0_ref[...],
                    preferred_element_type=jnp.float32)
            + jnp.dot(a1_ref[...] * r, wn1_ref[...],
                      preferred_element_type=jnp.float32)
            + jnp.dot(x_ref[...], ws_ref[...],
                      preferred_element_type=jnp.float32)
        )

    return pl.pallas_call(
        body,
        grid=(n // blk,),
        in_specs=[
            pl.BlockSpec((blk, HALF), lambda i: (i, 0)),
            pl.BlockSpec((blk, HALF), lambda i: (i, 0)),
            pl.BlockSpec((blk, HALF), lambda i: (i, 0)),
            pl.BlockSpec((blk, HALF), lambda i: (i, 0)),
            pl.BlockSpec((blk, d), lambda i: (i, 0)),
            pl.BlockSpec((HALF, d), lambda i: (0, 0)),
            pl.BlockSpec((HALF, d), lambda i: (0, 0)),
            pl.BlockSpec((d, d), lambda i: (0, 0)),
        ],
        out_specs=pl.BlockSpec((blk, d), lambda i: (i, 0)),
        out_shape=jax.ShapeDtypeStruct((n, d), jnp.float32),
    )(agg0, agg1, dega, degb, x_dst, wn0, wn1, w_self)


def _pad_edges(e_src, e_dst, n_dst):
    e = e_src.shape[0]
    chunk = NUM_SUBCORES * BLK * CHUNK
    e_pad = ((e + chunk - 1) // chunk) * chunk
    npad = e_pad - e
    e_src = jnp.concatenate(
        [e_src.astype(jnp.int32), jnp.zeros((npad,), jnp.int32)])
    e_dst = jnp.concatenate(
        [e_dst.astype(jnp.int32), jnp.full((npad,), n_dst, jnp.int32)])
    return e_src.reshape(-1, BLK), e_dst.reshape(-1, BLK)


def _conv(x_src, x_dst, e_src, e_dst, w_nbr, w_self):
    n_dst = x_dst.shape[0]
    es2d, ed2d = _pad_edges(e_src, e_dst, n_dst)
    x0 = x_src[:, :HALF]
    x1 = x_src[:, HALF:]
    agg0, agg1 = _sc_gather_scatter(x0, x1, es2d, ed2d, n_dst)
    dega, degb = _sc_degree(ed2d, n_dst)
    return _tc_mean_matmul(agg0, agg1, dega, degb, x_dst, w_nbr, w_self)


def kernel(x_user, x_item, u2i_src, u2i_dst, i2u_src, i2u_dst,
           n_user, n_item, W_nbr_u2i, W_self_u2i, W_nbr_i2u, W_self_i2u):
    out_item = _conv(x_user, x_item, u2i_src, u2i_dst, W_nbr_u2i, W_self_u2i)
    out_user = _conv(x_item, x_user, i2u_src, i2u_dst, W_nbr_i2u, W_self_i2u)
    return (out_user, out_item)
